# baseline (speedup 1.0000x reference)
import functools

import jax
import jax.numpy as jnp
from jax import lax
from jax.experimental import pallas as pl
from jax.experimental.pallas import tpu as pltpu
from jax.experimental.pallas import tpu_sc as plsc

N = 10000
NPAD = 10240
E = 320000
D = 128
P = 4096
NC = 2
NS = 16
NW = NC * NS
C = 128
CPW = 80
NCHUNKP = NW * CPW
EPAD = NCHUNKP * C
DEGW = 16
RPS = NPAD // NS
PPW = P // NW
BN = 640

_MESH = plsc.VectorSubcoreMesh(core_axis_name="c", subcore_axis_name="s")


def _make_sc_agg():
    out_type = jax.ShapeDtypeStruct((NC, NPAD, D), jnp.float32)
    scratch = (
        pltpu.VMEM((C,), jnp.int32),
        pltpu.VMEM((C,), jnp.int32),
        pltpu.VMEM((C,), jnp.int32),
        pltpu.VMEM((C,), jnp.int32),
        pltpu.VMEM((C, D), jnp.float32),
        pltpu.VMEM((C, D), jnp.float32),
        pltpu.VMEM_SHARED((NPAD, D), jnp.float32),
        pltpu.SemaphoreType.DMA,
        pltpu.SemaphoreType.DMA,
    )

    def body(x_hbm, src_hbm, dst_hbm, acc_out, src0_v, dst0_v, src1_v,
             dst1_v, rows0_v, rows1_v, acc_sh, sem0, sem1):
        cid = lax.axis_index("c")
        sid = lax.axis_index("s")
        wid = sid * NC + cid
        z = jnp.zeros((16,), jnp.float32)
        slots = ((src0_v, dst0_v, rows0_v, sem0),
                 (src1_v, dst1_v, rows1_v, sem1))

        def zero_rows(r, carry):
            for j in range(D // 16):
                rows0_v[r, pl.ds(j * 16, 16)] = z
            return carry

        lax.fori_loop(0, C, zero_rows, 0)
        for k in range(RPS // C):
            pltpu.sync_copy(rows0_v, acc_sh.at[pl.ds(sid * RPS + k * C, C)])

        def load_idx(c, src_b, dst_b):
            e0 = (wid * CPW + c) * C
            pltpu.sync_copy(src_hbm.at[pl.ds(e0, C)], src_b)
            pltpu.sync_copy(dst_hbm.at[pl.ds(e0, C)], dst_b)

        plsc.subcore_barrier()

        def gbody(c, carry):
            src_b, dst_b, rows_b, sem_b = slots[0]
            load_idx(c, src_b, dst_b)
            pltpu.async_copy(x_hbm.at[src_b], rows_b, sem_b).wait()
            pltpu.sync_copy(rows_b, acc_sh.at[dst_b], add=True)
            return carry

        lax.fori_loop(0, CPW, gbody, 0)
        plsc.subcore_barrier()

        pltpu.sync_copy(acc_sh.at[pl.ds(sid * RPS, RPS)],
                        acc_out.at[cid, pl.ds(sid * RPS, RPS)])

    return pl.kernel(body, out_type=out_type, mesh=_MESH,
                     scratch_types=scratch)


_sc_agg = _make_sc_agg()


@functools.partial(
    pl.kernel,
    out_type=jax.ShapeDtypeStruct((NC, NPAD, D), jnp.float32),
    mesh=_MESH,
    scratch_types=(
        pltpu.VMEM((CPW, C), jnp.int32),
        pltpu.VMEM((C,), jnp.int32),
        pltpu.VMEM((C, D), jnp.float32),
        pltpu.VMEM_SHARED((NPAD, D), jnp.float32),
    ),
)
def _sc_deg(dst_hbm, deg_out, dsts_v, dst_v, ones_v, acc_sh):
    cid = lax.axis_index("c")
    sid = lax.axis_index("s")
    wid = sid * NC + cid
    z = jnp.zeros((16,), jnp.float32)
    one = jnp.ones((16,), jnp.float32)

    def zero_rows(r, carry):
        for j in range(D // 16):
            ones_v[r, pl.ds(j * 16, 16)] = z
        return carry

    lax.fori_loop(0, C, zero_rows, 0)
    for k in range(RPS // C):
        pltpu.sync_copy(ones_v, acc_sh.at[pl.ds(sid * RPS + k * C, C)])

    def fill_ones(r, carry):
        for j in range(D // 16):
            ones_v[r, pl.ds(j * 16, 16)] = one
        return carry

    lax.fori_loop(0, C, fill_ones, 0)
    pltpu.sync_copy(dst_hbm.at[pl.ds(wid * CPW, CPW)], dsts_v)
    plsc.subcore_barrier()

    def ebody(c, carry):
        for j in range(C // 16):
            dst_v[pl.ds(j * 16, 16)] = dsts_v[c, pl.ds(j * 16, 16)]
        pltpu.sync_copy(ones_v, acc_sh.at[dst_v], add=True)
        return carry

    lax.fori_loop(0, CPW, ebody, 0)
    plsc.subcore_barrier()

    pltpu.sync_copy(acc_sh.at[pl.ds(sid * RPS, RPS)],
                    deg_out.at[cid, pl.ds(sid * RPS, RPS)])


def _tc_layer1(pacc, pdeg, W, b):
    def body(pacc_ref, pdeg_ref, w_ref, b_ref, x_ref, inv_ref):
        a = pacc_ref[0] + pacc_ref[1]
        dg = pdeg_ref[0][:, 0:1] + pdeg_ref[1][:, 0:1]
        inv = 1.0 / jnp.maximum(dg, 1.0)
        y = jnp.dot(a * inv, w_ref[...], preferred_element_type=jnp.float32)
        x_ref[...] = jnp.maximum(y + b_ref[...], 0.0)
        inv_ref[...] = jnp.broadcast_to(inv, (BN, DEGW))

    return pl.pallas_call(
        body,
        grid=(NPAD // BN,),
        in_specs=[
            pl.BlockSpec((NC, BN, D), lambda i: (0, i, 0)),
            pl.BlockSpec((NC, BN, D), lambda i: (0, i, 0)),
            pl.BlockSpec((D, D), lambda i: (0, 0)),
            pl.BlockSpec((1, D), lambda i: (0, 0)),
        ],
        out_specs=[
            pl.BlockSpec((BN, D), lambda i: (i, 0)),
            pl.BlockSpec((BN, DEGW), lambda i: (i, 0)),
        ],
        out_shape=[
            jax.ShapeDtypeStruct((NPAD, D), jnp.float32),
            jax.ShapeDtypeStruct((NPAD, DEGW), jnp.float32),
        ],
    )(pacc, pdeg, W, b)


def _tc_layer2(pacc, inv, W, b):
    def body(pacc_ref, inv_ref, w_ref, b_ref, x_ref):
        a = pacc_ref[0] + pacc_ref[1]
        y = jnp.dot(a * inv_ref[:, 0:1], w_ref[...],
                    preferred_element_type=jnp.float32)
        x_ref[...] = jnp.maximum(y + b_ref[...], 0.0)

    return pl.pallas_call(
        body,
        grid=(NPAD // BN,),
        in_specs=[
            pl.BlockSpec((NC, BN, D), lambda i: (0, i, 0)),
            pl.BlockSpec((BN, DEGW), lambda i: (i, 0)),
            pl.BlockSpec((D, D), lambda i: (0, 0)),
            pl.BlockSpec((1, D), lambda i: (0, 0)),
        ],
        out_specs=pl.BlockSpec((BN, D), lambda i: (i, 0)),
        out_shape=jax.ShapeDtypeStruct((NPAD, D), jnp.float32),
    )(pacc, inv, W, b)


@functools.partial(
    pl.kernel,
    out_type=(jax.ShapeDtypeStruct((P, D), jnp.float32),
              jax.ShapeDtypeStruct((P, D), jnp.float32)),
    mesh=_MESH,
    scratch_types=(
        pltpu.VMEM((PPW,), jnp.int32),
        pltpu.VMEM((PPW,), jnp.int32),
        pltpu.VMEM((PPW, D), jnp.float32),
        pltpu.VMEM((PPW, D), jnp.float32),
        pltpu.SemaphoreType.DMA,
    ),
)
def _sc_gather_pairs(x_hbm, s_hbm, d_hbm, xs_out, xd_out, si_v, di_v,
                     xs_v, xd_v, sem):
    cid = lax.axis_index("c")
    sid = lax.axis_index("s")
    wid = sid * NC + cid
    base = wid * PPW
    pltpu.sync_copy(s_hbm.at[pl.ds(base, PPW)], si_v)
    pltpu.sync_copy(d_hbm.at[pl.ds(base, PPW)], di_v)
    pltpu.async_copy(x_hbm.at[si_v], xs_v, sem).wait()
    pltpu.async_copy(x_hbm.at[di_v], xd_v, sem).wait()
    pltpu.sync_copy(xs_v, xs_out.at[pl.ds(base, PPW)])
    pltpu.sync_copy(xd_v, xd_out.at[pl.ds(base, PPW)])


def _tc_dot(xs, xd):
    def body(xs_ref, xd_ref, o_ref):
        o_ref[...] = jnp.sum(xs_ref[...] * xd_ref[...], axis=1, keepdims=True)

    return pl.pallas_call(
        body,
        out_shape=jax.ShapeDtypeStruct((P, 1), jnp.float32),
    )(xs, xd)


def kernel(edge_index, pair_index, emb, W1, b1, W2, b2):
    src = edge_index[0].astype(jnp.int32)
    dst = edge_index[1].astype(jnp.int32)
    ps = pair_index[0].astype(jnp.int32)
    pd = pair_index[1].astype(jnp.int32)
    npadlen = EPAD - E
    pad_dst = N + (jnp.arange(npadlen, dtype=jnp.int32) % (NPAD - N))
    dst_p = jnp.concatenate([dst, pad_dst])
    src_p = jnp.concatenate([src, jnp.zeros((npadlen,), jnp.int32)])
    dst2d = dst_p.reshape(NCHUNKP, C)
    x0 = jnp.zeros((NPAD, D), jnp.float32).at[:N].set(emb)
    pdeg = _sc_deg(dst2d)
    pacc1 = _sc_agg(x0, src_p, dst_p)
    x1, inv = _tc_layer1(pacc1, pdeg, W1, b1.reshape(1, D))
    pacc2 = _sc_agg(x1, src_p, dst_p)
    x2 = _tc_layer2(pacc2, inv, W2, b2.reshape(1, D))
    xs, xd = _sc_gather_pairs(x2, ps, pd)
    return _tc_dot(xs, xd).reshape(P)


# speedup vs baseline: 2.1933x; 2.1933x over previous
import functools

import jax
import jax.numpy as jnp
from jax import lax
from jax.experimental import pallas as pl
from jax.experimental.pallas import tpu as pltpu
from jax.experimental.pallas import tpu_sc as plsc

N = 10000
NPAD = 10240
E = 320000
D = 128
P = 4096
NC = 2
NS = 16
NW = NC * NS
C = 128
CPW = 80
NCHUNKP = NW * CPW
EPAD = NCHUNKP * C
DEGW = 16
RPS = NPAD // NS
PPW = P // NW
BN = 640

_MESH = plsc.VectorSubcoreMesh(core_axis_name="c", subcore_axis_name="s")


def _make_sc_agg():
    out_type = jax.ShapeDtypeStruct((NC, NPAD, D), jnp.float32)
    scratch = (
        pltpu.VMEM((C,), jnp.int32),
        pltpu.VMEM((C,), jnp.int32),
        pltpu.VMEM((C, D), jnp.float32),
        pltpu.VMEM_SHARED((NPAD, D), jnp.float32),
        pltpu.SemaphoreType.DMA,
    )

    def body(x_hbm, src_hbm, dst_hbm, acc_out, src0_v, dst0_v, rows0_v,
             acc_sh, sem0):
        cid = lax.axis_index("c")
        sid = lax.axis_index("s")
        wid = sid * NC + cid
        z = jnp.zeros((16,), jnp.float32)
        slots = ((src0_v, dst0_v, rows0_v, sem0),)

        def zero_rows(r, carry):
            for j in range(D // 16):
                rows0_v[r, pl.ds(j * 16, 16)] = z
            return carry

        lax.fori_loop(0, C, zero_rows, 0)
        for k in range(RPS // C):
            pltpu.sync_copy(rows0_v, acc_sh.at[pl.ds(sid * RPS + k * C, C)])

        def load_idx(c, src_b, dst_b):
            e0 = (wid * CPW + c) * C
            pltpu.sync_copy(src_hbm.at[pl.ds(e0, C)], src_b)
            pltpu.sync_copy(dst_hbm.at[pl.ds(e0, C)], dst_b)

        plsc.subcore_barrier()

        def gbody(c, carry):
            src_b, dst_b, rows_b, sem_b = slots[0]
            load_idx(c, src_b, dst_b)
            pltpu.async_copy(x_hbm.at[src_b], rows_b, sem_b).wait()
            pltpu.sync_copy(rows_b, acc_sh.at[dst_b], add=True)
            return carry

        lax.fori_loop(0, CPW, gbody, 0)
        plsc.subcore_barrier()

        pltpu.sync_copy(acc_sh.at[pl.ds(sid * RPS, RPS)],
                        acc_out.at[cid, pl.ds(sid * RPS, RPS)])

    return pl.kernel(body, out_type=out_type, mesh=_MESH,
                     scratch_types=scratch)


_sc_agg = _make_sc_agg()


@functools.partial(
    pl.kernel,
    out_type=jax.ShapeDtypeStruct((NC, NPAD, D), jnp.float32),
    mesh=_MESH,
    scratch_types=(
        pltpu.VMEM((CPW, C), jnp.int32),
        pltpu.VMEM((C,), jnp.int32),
        pltpu.VMEM((C, D), jnp.float32),
        pltpu.VMEM_SHARED((NPAD, D), jnp.float32),
    ),
)
def _sc_deg(dst_hbm, deg_out, dsts_v, dst_v, ones_v, acc_sh):
    cid = lax.axis_index("c")
    sid = lax.axis_index("s")
    wid = sid * NC + cid
    z = jnp.zeros((16,), jnp.float32)
    one = jnp.ones((16,), jnp.float32)

    def zero_rows(r, carry):
        for j in range(D // 16):
            ones_v[r, pl.ds(j * 16, 16)] = z
        return carry

    lax.fori_loop(0, C, zero_rows, 0)
    for k in range(RPS // C):
        pltpu.sync_copy(ones_v, acc_sh.at[pl.ds(sid * RPS + k * C, C)])

    def fill_ones(r, carry):
        for j in range(D // 16):
            ones_v[r, pl.ds(j * 16, 16)] = one
        return carry

    lax.fori_loop(0, C, fill_ones, 0)
    pltpu.sync_copy(dst_hbm.at[pl.ds(wid * CPW, CPW)], dsts_v)
    plsc.subcore_barrier()

    def ebody(c, carry):
        for j in range(C // 16):
            dst_v[pl.ds(j * 16, 16)] = dsts_v[c, pl.ds(j * 16, 16)]
        pltpu.sync_copy(ones_v, acc_sh.at[dst_v], add=True)
        return carry

    lax.fori_loop(0, CPW, ebody, 0)
    plsc.subcore_barrier()

    pltpu.sync_copy(acc_sh.at[pl.ds(sid * RPS, RPS)],
                    deg_out.at[cid, pl.ds(sid * RPS, RPS)])


def _tc_layer1(pacc, pdeg, W, b):
    def body(pacc_ref, pdeg_ref, w_ref, b_ref, x_ref, inv_ref):
        a = pacc_ref[0] + pacc_ref[1]
        dg = pdeg_ref[0][:, 0:1] + pdeg_ref[1][:, 0:1]
        inv = 1.0 / jnp.maximum(dg, 1.0)
        y = jnp.dot(a * inv, w_ref[...], preferred_element_type=jnp.float32)
        x_ref[...] = jnp.maximum(y + b_ref[...], 0.0)
        inv_ref[...] = jnp.broadcast_to(inv, (BN, DEGW))

    return pl.pallas_call(
        body,
        grid=(NPAD // BN,),
        in_specs=[
            pl.BlockSpec((NC, BN, D), lambda i: (0, i, 0)),
            pl.BlockSpec((NC, BN, D), lambda i: (0, i, 0)),
            pl.BlockSpec((D, D), lambda i: (0, 0)),
            pl.BlockSpec((1, D), lambda i: (0, 0)),
        ],
        out_specs=[
            pl.BlockSpec((BN, D), lambda i: (i, 0)),
            pl.BlockSpec((BN, DEGW), lambda i: (i, 0)),
        ],
        out_shape=[
            jax.ShapeDtypeStruct((NPAD, D), jnp.float32),
            jax.ShapeDtypeStruct((NPAD, DEGW), jnp.float32),
        ],
    )(pacc, pdeg, W, b)


def _tc_layer2(pacc, inv, W, b):
    def body(pacc_ref, inv_ref, w_ref, b_ref, x_ref):
        a = pacc_ref[0] + pacc_ref[1]
        y = jnp.dot(a * inv_ref[:, 0:1], w_ref[...],
                    preferred_element_type=jnp.float32)
        x_ref[...] = jnp.maximum(y + b_ref[...], 0.0)

    return pl.pallas_call(
        body,
        grid=(NPAD // BN,),
        in_specs=[
            pl.BlockSpec((NC, BN, D), lambda i: (0, i, 0)),
            pl.BlockSpec((BN, DEGW), lambda i: (i, 0)),
            pl.BlockSpec((D, D), lambda i: (0, 0)),
            pl.BlockSpec((1, D), lambda i: (0, 0)),
        ],
        out_specs=pl.BlockSpec((BN, D), lambda i: (i, 0)),
        out_shape=jax.ShapeDtypeStruct((NPAD, D), jnp.float32),
    )(pacc, inv, W, b)


@functools.partial(
    pl.kernel,
    out_type=(jax.ShapeDtypeStruct((P, D), jnp.float32),
              jax.ShapeDtypeStruct((P, D), jnp.float32)),
    mesh=_MESH,
    scratch_types=(
        pltpu.VMEM((PPW,), jnp.int32),
        pltpu.VMEM((PPW,), jnp.int32),
        pltpu.VMEM((PPW, D), jnp.float32),
        pltpu.VMEM((PPW, D), jnp.float32),
        pltpu.SemaphoreType.DMA,
    ),
)
def _sc_gather_pairs(x_hbm, s_hbm, d_hbm, xs_out, xd_out, si_v, di_v,
                     xs_v, xd_v, sem):
    cid = lax.axis_index("c")
    sid = lax.axis_index("s")
    wid = sid * NC + cid
    base = wid * PPW
    pltpu.sync_copy(s_hbm.at[pl.ds(base, PPW)], si_v)
    pltpu.sync_copy(d_hbm.at[pl.ds(base, PPW)], di_v)
    pltpu.async_copy(x_hbm.at[si_v], xs_v, sem).wait()
    pltpu.async_copy(x_hbm.at[di_v], xd_v, sem).wait()
    pltpu.sync_copy(xs_v, xs_out.at[pl.ds(base, PPW)])
    pltpu.sync_copy(xd_v, xd_out.at[pl.ds(base, PPW)])


def _tc_dot(xs, xd):
    def body(xs_ref, xd_ref, o_ref):
        o_ref[...] = jnp.sum(xs_ref[...] * xd_ref[...], axis=1, keepdims=True)

    return pl.pallas_call(
        body,
        out_shape=jax.ShapeDtypeStruct((P, 1), jnp.float32),
    )(xs, xd)


def kernel(edge_index, pair_index, emb, W1, b1, W2, b2):
    src = edge_index[0].astype(jnp.int32)
    dst = edge_index[1].astype(jnp.int32)
    ps = pair_index[0].astype(jnp.int32)
    pd = pair_index[1].astype(jnp.int32)
    npadlen = EPAD - E
    pad_iota = jnp.arange(npadlen, dtype=jnp.int32)
    pad_dst = N + pad_iota % (NPAD - N)
    dst_p = jnp.concatenate([dst, pad_dst])
    src_p = jnp.concatenate([src, pad_iota % N])
    dst2d = dst_p.reshape(NCHUNKP, C)
    x0 = jnp.zeros((NPAD, D), jnp.float32).at[:N].set(emb)
    pdeg = _sc_deg(dst2d)
    pacc1 = _sc_agg(x0, src_p, dst_p)
    x1, inv = _tc_layer1(pacc1, pdeg, W1, b1.reshape(1, D))
    pacc2 = _sc_agg(x1, src_p, dst_p)
    x2 = _tc_layer2(pacc2, inv, W2, b2.reshape(1, D))
    xs, xd = _sc_gather_pairs(x2, ps, pd)
    return _tc_dot(xs, xd).reshape(P)


# speedup vs baseline: 3.1623x; 1.4418x over previous
import functools

import jax
import jax.numpy as jnp
from jax import lax
from jax.experimental import pallas as pl
from jax.experimental.pallas import tpu as pltpu
from jax.experimental.pallas import tpu_sc as plsc

N = 10000
NPAD = 10240
E = 320000
D = 128
P = 4096
NC = 2
NS = 16
NW = NC * NS
C = 128
CPW = 80
NCHUNKP = NW * CPW
EPAD = NCHUNKP * C
DEGW = 16
RPS = NPAD // NS
PPW = P // NW
BN = 640

_MESH = plsc.VectorSubcoreMesh(core_axis_name="c", subcore_axis_name="s")


def _make_sc_agg():
    out_type = jax.ShapeDtypeStruct((NC, NPAD, D), jnp.float32)
    scratch = (
        pltpu.VMEM((C,), jnp.int32),
        pltpu.VMEM((C,), jnp.int32),
        pltpu.VMEM((C,), jnp.int32),
        pltpu.VMEM((C,), jnp.int32),
        pltpu.VMEM((C, D), jnp.float32),
        pltpu.VMEM((C, D), jnp.float32),
        pltpu.VMEM_SHARED((NPAD, D), jnp.float32),
        pltpu.SemaphoreType.DMA,
        pltpu.SemaphoreType.DMA,
    )

    def body(x_hbm, src_hbm, dst_hbm, acc_out, src0_v, dst0_v, src1_v,
             dst1_v, rows0_v, rows1_v, acc_sh, sem0, sem1):
        cid = lax.axis_index("c")
        sid = lax.axis_index("s")
        wid = sid * NC + cid
        z = jnp.zeros((16,), jnp.float32)
        slots = ((src0_v, dst0_v, rows0_v, sem0),
                 (src1_v, dst1_v, rows1_v, sem1))

        def zero_rows(r, carry):
            for j in range(D // 16):
                rows0_v[r, pl.ds(j * 16, 16)] = z
            return carry

        lax.fori_loop(0, C, zero_rows, 0)
        for k in range(RPS // C):
            pltpu.sync_copy(rows0_v, acc_sh.at[pl.ds(sid * RPS + k * C, C)])

        def load_idx(c, src_b, dst_b):
            e0 = (wid * CPW + c) * C
            pltpu.sync_copy(src_hbm.at[pl.ds(e0, C)], src_b)
            pltpu.sync_copy(dst_hbm.at[pl.ds(e0, C)], dst_b)

        plsc.subcore_barrier()

        for b in (0, 1):
            src_b, dst_b, rows_b, sem_b = slots[b]
            load_idx(b, src_b, dst_b)
            pltpu.async_copy(x_hbm.at[src_b], rows_b, sem_b)

        def gbody(g, carry):
            for b in (0, 1):
                c = 2 * g + b
                src_b, dst_b, rows_b, sem_b = slots[b]
                pltpu.make_async_copy(x_hbm.at[src_b], rows_b, sem_b).wait()
                pltpu.sync_copy(rows_b, acc_sh.at[dst_b], add=True)

                @pl.when(c + 2 < CPW)
                def _():
                    load_idx(c + 2, src_b, dst_b)
                    pltpu.async_copy(x_hbm.at[src_b], rows_b, sem_b)
            return carry

        lax.fori_loop(0, CPW // 2, gbody, 0)
        plsc.subcore_barrier()

        pltpu.sync_copy(acc_sh.at[pl.ds(sid * RPS, RPS)],
                        acc_out.at[cid, pl.ds(sid * RPS, RPS)])

    return pl.kernel(body, out_type=out_type, mesh=_MESH,
                     scratch_types=scratch)


_sc_agg = _make_sc_agg()


@functools.partial(
    pl.kernel,
    out_type=jax.ShapeDtypeStruct((NC, NPAD, D), jnp.float32),
    mesh=_MESH,
    scratch_types=(
        pltpu.VMEM((CPW, C), jnp.int32),
        pltpu.VMEM((C,), jnp.int32),
        pltpu.VMEM((C, D), jnp.float32),
        pltpu.VMEM_SHARED((NPAD, D), jnp.float32),
    ),
)
def _sc_deg(dst_hbm, deg_out, dsts_v, dst_v, ones_v, acc_sh):
    cid = lax.axis_index("c")
    sid = lax.axis_index("s")
    wid = sid * NC + cid
    z = jnp.zeros((16,), jnp.float32)
    one = jnp.ones((16,), jnp.float32)

    def zero_rows(r, carry):
        for j in range(D // 16):
            ones_v[r, pl.ds(j * 16, 16)] = z
        return carry

    lax.fori_loop(0, C, zero_rows, 0)
    for k in range(RPS // C):
        pltpu.sync_copy(ones_v, acc_sh.at[pl.ds(sid * RPS + k * C, C)])

    def fill_ones(r, carry):
        for j in range(D // 16):
            ones_v[r, pl.ds(j * 16, 16)] = one
        return carry

    lax.fori_loop(0, C, fill_ones, 0)
    pltpu.sync_copy(dst_hbm.at[pl.ds(wid * CPW, CPW)], dsts_v)
    plsc.subcore_barrier()

    def ebody(c, carry):
        for j in range(C // 16):
            dst_v[pl.ds(j * 16, 16)] = dsts_v[c, pl.ds(j * 16, 16)]
        pltpu.sync_copy(ones_v, acc_sh.at[dst_v], add=True)
        return carry

    lax.fori_loop(0, CPW, ebody, 0)
    plsc.subcore_barrier()

    pltpu.sync_copy(acc_sh.at[pl.ds(sid * RPS, RPS)],
                    deg_out.at[cid, pl.ds(sid * RPS, RPS)])


def _tc_layer1(pacc, pdeg, W, b):
    def body(pacc_ref, pdeg_ref, w_ref, b_ref, x_ref, inv_ref):
        a = pacc_ref[0] + pacc_ref[1]
        dg = pdeg_ref[0][:, 0:1] + pdeg_ref[1][:, 0:1]
        inv = 1.0 / jnp.maximum(dg, 1.0)
        y = jnp.dot(a * inv, w_ref[...], preferred_element_type=jnp.float32)
        x_ref[...] = jnp.maximum(y + b_ref[...], 0.0)
        inv_ref[...] = jnp.broadcast_to(inv, (BN, DEGW))

    return pl.pallas_call(
        body,
        grid=(NPAD // BN,),
        in_specs=[
            pl.BlockSpec((NC, BN, D), lambda i: (0, i, 0)),
            pl.BlockSpec((NC, BN, D), lambda i: (0, i, 0)),
            pl.BlockSpec((D, D), lambda i: (0, 0)),
            pl.BlockSpec((1, D), lambda i: (0, 0)),
        ],
        out_specs=[
            pl.BlockSpec((BN, D), lambda i: (i, 0)),
            pl.BlockSpec((BN, DEGW), lambda i: (i, 0)),
        ],
        out_shape=[
            jax.ShapeDtypeStruct((NPAD, D), jnp.float32),
            jax.ShapeDtypeStruct((NPAD, DEGW), jnp.float32),
        ],
    )(pacc, pdeg, W, b)


def _tc_layer2(pacc, inv, W, b):
    def body(pacc_ref, inv_ref, w_ref, b_ref, x_ref):
        a = pacc_ref[0] + pacc_ref[1]
        y = jnp.dot(a * inv_ref[:, 0:1], w_ref[...],
                    preferred_element_type=jnp.float32)
        x_ref[...] = jnp.maximum(y + b_ref[...], 0.0)

    return pl.pallas_call(
        body,
        grid=(NPAD // BN,),
        in_specs=[
            pl.BlockSpec((NC, BN, D), lambda i: (0, i, 0)),
            pl.BlockSpec((BN, DEGW), lambda i: (i, 0)),
            pl.BlockSpec((D, D), lambda i: (0, 0)),
            pl.BlockSpec((1, D), lambda i: (0, 0)),
        ],
        out_specs=pl.BlockSpec((BN, D), lambda i: (i, 0)),
        out_shape=jax.ShapeDtypeStruct((NPAD, D), jnp.float32),
    )(pacc, inv, W, b)


@functools.partial(
    pl.kernel,
    out_type=(jax.ShapeDtypeStruct((P, D), jnp.float32),
              jax.ShapeDtypeStruct((P, D), jnp.float32)),
    mesh=_MESH,
    scratch_types=(
        pltpu.VMEM((PPW,), jnp.int32),
        pltpu.VMEM((PPW,), jnp.int32),
        pltpu.VMEM((PPW, D), jnp.float32),
        pltpu.VMEM((PPW, D), jnp.float32),
        pltpu.SemaphoreType.DMA,
    ),
)
def _sc_gather_pairs(x_hbm, s_hbm, d_hbm, xs_out, xd_out, si_v, di_v,
                     xs_v, xd_v, sem):
    cid = lax.axis_index("c")
    sid = lax.axis_index("s")
    wid = sid * NC + cid
    base = wid * PPW
    pltpu.sync_copy(s_hbm.at[pl.ds(base, PPW)], si_v)
    pltpu.sync_copy(d_hbm.at[pl.ds(base, PPW)], di_v)
    pltpu.async_copy(x_hbm.at[si_v], xs_v, sem).wait()
    pltpu.async_copy(x_hbm.at[di_v], xd_v, sem).wait()
    pltpu.sync_copy(xs_v, xs_out.at[pl.ds(base, PPW)])
    pltpu.sync_copy(xd_v, xd_out.at[pl.ds(base, PPW)])


def _tc_dot(xs, xd):
    def body(xs_ref, xd_ref, o_ref):
        o_ref[...] = jnp.sum(xs_ref[...] * xd_ref[...], axis=1, keepdims=True)

    return pl.pallas_call(
        body,
        out_shape=jax.ShapeDtypeStruct((P, 1), jnp.float32),
    )(xs, xd)


def kernel(edge_index, pair_index, emb, W1, b1, W2, b2):
    src = edge_index[0].astype(jnp.int32)
    dst = edge_index[1].astype(jnp.int32)
    ps = pair_index[0].astype(jnp.int32)
    pd = pair_index[1].astype(jnp.int32)
    npadlen = EPAD - E
    pad_iota = jnp.arange(npadlen, dtype=jnp.int32)
    pad_dst = N + pad_iota % (NPAD - N)
    dst_p = jnp.concatenate([dst, pad_dst])
    src_p = jnp.concatenate([src, pad_iota % N])
    dst2d = dst_p.reshape(NCHUNKP, C)
    x0 = jnp.zeros((NPAD, D), jnp.float32).at[:N].set(emb)
    pdeg = _sc_deg(dst2d)
    pacc1 = _sc_agg(x0, src_p, dst_p)
    x1, inv = _tc_layer1(pacc1, pdeg, W1, b1.reshape(1, D))
    pacc2 = _sc_agg(x1, src_p, dst_p)
    x2 = _tc_layer2(pacc2, inv, W2, b2.reshape(1, D))
    xs, xd = _sc_gather_pairs(x2, ps, pd)
    return _tc_dot(xs, xd).reshape(P)
